# Initial kernel scaffold; baseline (speedup 1.0000x reference)
import jax
import jax.numpy as jnp
from jax.experimental import pallas as pl


def kernel(image_features, desired_q, desired_k):
    raise NotImplementedError("write your pallas kernel here")


# speedup vs baseline: 3.5902x; 3.5902x over previous
import jax
import jax.numpy as jnp
from jax.experimental import pallas as pl
from jax.experimental.pallas import tpu as pltpu

B, N, C = 8, 576, 1024
LEFT = 72
CLUSTER_K = 32
BIG = 1e9


def _kernel(q_ref, kcls_ref, kk_ref, x_ref, out_ref, g_ref):
    f32 = jnp.float32
    q2 = q_ref[0]
    kcls = kcls_ref[0]
    kk = kk_ref[0]
    x = x_ref[0]

    lk = jax.lax.dot_general(q2, kk, (((1,), (1,)), ((), ())),
                             preferred_element_type=f32)
    lcls = jnp.sum(q2 * kcls)
    scale = f32(C) ** f32(-0.5)
    lk = lk * scale
    lcls = lcls * scale
    m = jnp.maximum(jnp.max(lk), lcls)
    ek = jnp.exp(lk - m)
    s = jnp.sum(ek) + jnp.exp(lcls - m)
    attn = ek / s

    kk2 = kk * kk
    ones_row = jnp.ones((1, C), dtype=f32)
    nsq = jax.lax.dot_general(ones_row, kk2, (((1,), (1,)), ((), ())),
                              preferred_element_type=f32)
    invn = 1.0 / jnp.maximum(jnp.sqrt(nsq), f32(1e-12))

    iota_row = jax.lax.broadcasted_iota(jnp.int32, (1, N), 1)
    ibig = jnp.int32(2 ** 30)

    g_ref[...] = jnp.zeros((80, N), dtype=f32)

    def top72_body(i, carry):
        arr, S = carry
        mx = jnp.max(arr)
        cand = jnp.where(arr == mx, iota_row, ibig)
        sel = jnp.min(cand)
        oh = (iota_row == sel).astype(f32)
        g_ref[pl.ds(i, 1), :] = oh
        arr = jnp.where(oh > 0, f32(-1.0), arr)
        return arr, S + oh

    _, S = jax.lax.fori_loop(0, LEFT, top72_body,
                             (attn, jnp.zeros((1, N), dtype=f32)))

    P = g_ref[0:LEFT, :]

    ksel = jax.lax.dot_general(P, kk, (((1,), (0,)), ((), ())),
                               preferred_element_type=f32)
    invnsel = jax.lax.dot_general(P, invn, (((1,), (1,)), ((), ())),
                                  preferred_element_type=f32)
    cos = jax.lax.dot_general(ksel, kk, (((1,), (1,)), ((), ())),
                              preferred_element_type=f32)
    cos = cos * invnsel * invn
    cos = jnp.where(P > 0.5, f32(-3.0), cos)

    iota2 = jax.lax.broadcasted_iota(jnp.int32, (LEFT, N), 1)

    def top32_body(i, carry):
        cw, M = carry
        mx = jnp.max(cw, axis=1, keepdims=True)
        cand = jnp.where(cw == mx, iota2, ibig)
        sel = jnp.min(cand, axis=1, keepdims=True)
        oh = (iota2 == sel).astype(f32)
        return jnp.where(oh > 0, f32(-3.0), cw), M + oh

    _, M = jax.lax.fori_loop(0, CLUSTER_K, top32_body,
                             (cos, jnp.zeros((LEFT, N), dtype=f32)))

    g_ref[0:LEFT, :] = P + M * attn
    g_ref[LEFT:LEFT + 1, :] = (1.0 - S) * attn
    res = jax.lax.dot_general(g_ref[...], x, (((1,), (0,)), ((), ())),
                              preferred_element_type=f32)
    out_ref[...] = res[0:LEFT + 1, :][None]


def kernel(image_features, desired_q, desired_k):
    q0 = desired_q[:, 0:1, :]
    kcls = desired_k[:, 0:1, :]
    kk = desired_k[:, 1:, :]
    out = pl.pallas_call(
        _kernel,
        grid=(B,),
        in_specs=[
            pl.BlockSpec((1, 1, C), lambda b: (b, 0, 0)),
            pl.BlockSpec((1, 1, C), lambda b: (b, 0, 0)),
            pl.BlockSpec((1, N, C), lambda b: (b, 0, 0)),
            pl.BlockSpec((1, N, C), lambda b: (b, 0, 0)),
        ],
        out_specs=pl.BlockSpec((1, LEFT + 1, C), lambda b: (b, 0, 0)),
        out_shape=jax.ShapeDtypeStruct((B, LEFT + 1, C), jnp.float32),
        scratch_shapes=[pltpu.VMEM((80, N), jnp.float32)],
    )(q0, kcls, kk, image_features)
    return out


# speedup vs baseline: 11.8987x; 3.3142x over previous
import jax
import jax.numpy as jnp
from jax.experimental import pallas as pl
from jax.experimental.pallas import tpu as pltpu

B, N, C = 8, 576, 1024
LEFT = 72
CLUSTER_K = 32


def _kernel(q_ref, kcls_ref, kk_ref, x_ref, out_ref):
    f32 = jnp.float32
    i32 = jnp.int32
    q2 = q_ref[0]
    kcls = kcls_ref[0]
    kk = kk_ref[0]
    x = x_ref[0]

    lk = jax.lax.dot_general(q2, kk, (((1,), (1,)), ((), ())),
                             preferred_element_type=f32)
    lcls = jnp.sum(q2 * kcls)
    scale = f32(C) ** f32(-0.5)
    lk = lk * scale
    lcls = lcls * scale
    m = jnp.maximum(jnp.max(lk), lcls)
    ek = jnp.exp(lk - m)
    s = jnp.sum(ek) + jnp.exp(lcls - m)
    attn = ek / s
    attn_col = jnp.transpose(lk) * f32(1.0)
    attn_col = jnp.exp(attn_col - m) / s

    io_c = jax.lax.broadcasted_iota(i32, (N, N), 0)
    io_r = jax.lax.broadcasted_iota(i32, (N, N), 1)
    beats = (attn_col > attn) | ((attn_col == attn) & (io_c < io_r))
    rank = jnp.sum(beats.astype(i32), axis=0, keepdims=True)
    rank_rows = jax.lax.broadcasted_iota(i32, (LEFT, N), 0)
    P = (rank == rank_rows).astype(f32)
    S = (rank < LEFT).astype(f32)

    ones_row = jnp.ones((1, C), dtype=f32)
    nsq = jax.lax.dot_general(ones_row, kk * kk, (((1,), (1,)), ((), ())),
                              preferred_element_type=f32)
    invn = 1.0 / jnp.maximum(jnp.sqrt(nsq), f32(1e-12))

    ksel = jax.lax.dot_general(P, kk, (((1,), (0,)), ((), ())),
                               preferred_element_type=f32)
    invnsel = jax.lax.dot_general(P, invn, (((1,), (1,)), ((), ())),
                                  preferred_element_type=f32)
    cos = jax.lax.dot_general(ksel, kk, (((1,), (1,)), ((), ())),
                              preferred_element_type=f32)
    cos = cos * invnsel * invn
    cos = jnp.where(P > 0.5, f32(-3.0), cos)

    bits = jax.lax.bitcast_convert_type(cos, i32)
    skey = bits ^ (jax.lax.shift_right_arithmetic(bits, 31) & i32(0x7FFFFFFF))
    iota2 = jax.lax.broadcasted_iota(i32, (LEFT, N), 1)
    skey = (skey & i32(~1023)) | (i32(1023) - iota2)
    neg_inf_key = i32(-(2 ** 31) + 1)

    def top32_body(i, carry):
        kw, M = carry
        mx = jnp.max(kw, axis=1, keepdims=True)
        oh = kw == mx
        return jnp.where(oh, neg_inf_key, kw), M + oh.astype(f32)

    _, M = jax.lax.fori_loop(0, CLUSTER_K, top32_body,
                             (skey, jnp.zeros((LEFT, N), dtype=f32)),
                             unroll=True)

    res = jax.lax.dot_general(P + M * attn, x, (((1,), (0,)), ((), ())),
                              preferred_element_type=f32)
    extra = jax.lax.dot_general((1.0 - S) * attn, x, (((1,), (0,)), ((), ())),
                                preferred_element_type=f32)
    out_ref[0, 0:LEFT, :] = res
    out_ref[0, LEFT:LEFT + 1, :] = extra


def kernel(image_features, desired_q, desired_k):
    q0 = desired_q[:, 0:1, :]
    kcls = desired_k[:, 0:1, :]
    kk = desired_k[:, 1:, :]
    out = pl.pallas_call(
        _kernel,
        grid=(B,),
        in_specs=[
            pl.BlockSpec((1, 1, C), lambda b: (b, 0, 0)),
            pl.BlockSpec((1, 1, C), lambda b: (b, 0, 0)),
            pl.BlockSpec((1, N, C), lambda b: (b, 0, 0)),
            pl.BlockSpec((1, N, C), lambda b: (b, 0, 0)),
        ],
        out_specs=pl.BlockSpec((1, LEFT + 1, C), lambda b: (b, 0, 0)),
        out_shape=jax.ShapeDtypeStruct((B, LEFT + 1, C), jnp.float32),
        compiler_params=pltpu.CompilerParams(
            dimension_semantics=("parallel",)),
    )(q0, kcls, kk, image_features)
    return out
